# baseline (speedup 1.0000x reference)
import functools
import math

import jax
import jax.numpy as jnp
from jax import lax
from jax.experimental import pallas as pl
from jax.experimental.pallas import tpu as pltpu
from jax.experimental.pallas import tpu_sc as plsc

D = 64
SCALE = math.sqrt(D)
NUM_CORES = 2
NUM_SUBCORES = 16
NW = NUM_CORES * NUM_SUBCORES
LANES = 16
CHUNK = 256


@jax.jit
def _embed(tokens_flat, table_pairs):
    B = tokens_flat.shape[0]
    b_per_w = B // NW
    n_chunks = b_per_w // CHUNK
    mesh = plsc.VectorSubcoreMesh(core_axis_name="c", subcore_axis_name="s")

    @functools.partial(
        pl.kernel,
        mesh=mesh,
        out_type=jax.ShapeDtypeStruct((B // 2, 2 * D), jnp.float32),
        scratch_types=[
            pltpu.VMEM((CHUNK,), jnp.int32),
            pltpu.VMEM((CHUNK,), jnp.int32),
            pltpu.VMEM((CHUNK, 2 * D), jnp.float32),
            pltpu.VMEM((CHUNK // 2, 2 * D), jnp.float32),
            pltpu.SemaphoreType.DMA,
        ],
        compiler_params=pltpu.CompilerParams(
            use_tc_tiling_on_sc=True, needs_layout_passes=False
        ),
    )
    def k(tok_hbm, table_hbm, out_hbm, idx_v, pidx_v, gbuf, obuf, sem):
        wid = lax.axis_index("s") * NUM_CORES + lax.axis_index("c")
        base = wid * b_per_w
        iota = lax.iota(jnp.int32, LANES)

        def chunk_body(ci, carry):
            off = pl.multiple_of(base + ci * CHUNK, CHUNK)
            pltpu.sync_copy(tok_hbm.at[pl.ds(off, CHUNK)], idx_v)

            def mk_pidx(g, c):
                sl = pl.ds(g * LANES, LANES)
                pidx_v[sl] = lax.shift_right_logical(idx_v[sl], 1)
                return c

            lax.fori_loop(0, CHUNK // LANES, mk_pidx, 0)
            pltpu.async_copy(table_hbm.at[pidx_v], gbuf, sem).wait()

            def select_group(g, c):
                i0 = 16 * g
                tokv = idx_v[pl.ds(i0, LANES)]
                for l in range(LANES):
                    hoff = (tokv[l] & 1) * D
                    rowv = jnp.full((LANES,), i0 + l, jnp.int32)
                    orowv = jnp.full((LANES,), (i0 + l) // 2, jnp.int32)
                    half = l % 2
                    for j in range(D // LANES):
                        colv = iota + (hoff + j * LANES)
                        ocolv = iota + (half * D + j * LANES)
                        val = plsc.load_gather(gbuf, [rowv, colv])
                        plsc.store_scatter(obuf, [orowv, ocolv], val * SCALE)
                return c

            lax.fori_loop(0, CHUNK // LANES, select_group, 0)
            ooff = pl.multiple_of(off // 2, CHUNK // 2)
            pltpu.sync_copy(obuf, out_hbm.at[pl.ds(ooff, CHUNK // 2)])
            return carry

        lax.fori_loop(0, n_chunks, chunk_body, 0)

    return k(tokens_flat, table_pairs)


def kernel(tokens, table):
    BATCH, SEQ = tokens.shape
    B = BATCH * SEQ
    flat = tokens.reshape(B).astype(jnp.int32)
    out = _embed(flat, table.reshape(table.shape[0] // 2, 2 * D))
    return out.reshape(BATCH, SEQ, D)


# speedup vs baseline: 1.5489x; 1.5489x over previous
import functools
import math

import jax
import jax.numpy as jnp
from jax import lax
from jax.experimental import pallas as pl
from jax.experimental.pallas import tpu as pltpu
from jax.experimental.pallas import tpu_sc as plsc

D = 64
SCALE = math.sqrt(D)
NUM_CORES = 2
NUM_SUBCORES = 16
NW = NUM_CORES * NUM_SUBCORES
LANES = 16
CHUNK = 400
ROW_UNROLL = 8


def _scale_chunk(src, dst):
    def rows(i, c):
        for r in range(ROW_UNROLL):
            for j in range(D // LANES):
                sl = pl.ds(j * LANES, LANES)
                dst[i * ROW_UNROLL + r, sl] = src[i * ROW_UNROLL + r, sl] * SCALE
        return c

    lax.fori_loop(0, CHUNK // ROW_UNROLL, rows, 0, unroll=False)


@jax.jit
def _embed(tokens_flat, table):
    B = tokens_flat.shape[0]
    b_per_w = B // NW
    n_chunks = b_per_w // CHUNK
    mesh = plsc.VectorSubcoreMesh(core_axis_name="c", subcore_axis_name="s")

    @functools.partial(
        pl.kernel,
        mesh=mesh,
        out_type=jax.ShapeDtypeStruct((B, D), jnp.float32),
        scratch_types=[
            pltpu.VMEM((CHUNK,), jnp.int32),
            pltpu.VMEM((CHUNK,), jnp.int32),
            pltpu.VMEM((CHUNK, D), jnp.float32),
            pltpu.VMEM((CHUNK, D), jnp.float32),
            pltpu.VMEM((CHUNK, D), jnp.float32),
            pltpu.VMEM((CHUNK, D), jnp.float32),
            pltpu.SemaphoreType.DMA,
            pltpu.SemaphoreType.DMA,
            pltpu.SemaphoreType.DMA,
            pltpu.SemaphoreType.DMA,
        ],
        compiler_params=pltpu.CompilerParams(use_tc_tiling_on_sc=False),
    )
    def k(tok_hbm, table_hbm, out_hbm,
          idx0, idx1, g0, g1, o0, o1, gs0, gs1, ss0, ss1):
        idx = (idx0, idx1)
        gbuf = (g0, g1)
        obuf = (o0, o1)
        gsem = (gs0, gs1)
        ssem = (ss0, ss1)
        wid = lax.axis_index("s") * NUM_CORES + lax.axis_index("c")
        base = wid * b_per_w

        def start_gather(c, b):
            off = base + c * CHUNK
            pltpu.sync_copy(tok_hbm.at[pl.ds(off, CHUNK)], idx[b])
            pltpu.make_async_copy(table_hbm.at[idx[b]], gbuf[b], gsem[b]).start()

        def wait_gather(b):
            pltpu.make_async_copy(table_hbm.at[idx[b]], gbuf[b], gsem[b]).wait()

        def start_scatter(c, b):
            off = base + c * CHUNK
            pltpu.make_async_copy(obuf[b], out_hbm.at[pl.ds(off, CHUNK)],
                                  ssem[b]).start()

        def wait_scatter(c, b):
            off = base + c * CHUNK
            pltpu.make_async_copy(obuf[b], out_hbm.at[pl.ds(off, CHUNK)],
                                  ssem[b]).wait()

        start_gather(0, 0)
        start_gather(1, 1)
        for c in (0, 1):
            b = c & 1
            wait_gather(b)
            _scale_chunk(gbuf[b], obuf[b])
            start_scatter(c, b)
            start_gather(c + 2, b)

        def body(i, carry):
            for b in (0, 1):
                c = 2 + 2 * i + b
                wait_gather(b)
                wait_scatter(c - 2, b)
                _scale_chunk(gbuf[b], obuf[b])
                start_scatter(c, b)
                start_gather(c + 2, b)
            return carry

        lax.fori_loop(0, (n_chunks - 4) // 2, body, 0, unroll=False)

        for c in (n_chunks - 2, n_chunks - 1):
            b = c & 1
            wait_gather(b)
            wait_scatter(c - 2, b)
            _scale_chunk(gbuf[b], obuf[b])
            start_scatter(c, b)
        for c in (n_chunks - 2, n_chunks - 1):
            wait_scatter(c, c & 1)

    return k(tokens_flat, table)


def kernel(tokens, table):
    BATCH, SEQ = tokens.shape
    B = BATCH * SEQ
    flat = tokens.reshape(B).astype(jnp.int32)
    out = _embed(flat, table)
    return out.reshape(BATCH, SEQ, D)
